# Initial kernel scaffold; baseline (speedup 1.0000x reference)
import jax
import jax.numpy as jnp
from jax.experimental import pallas as pl


def kernel(x, codebooks):
    raise NotImplementedError("write your pallas kernel here")


# speedup vs baseline: 1.0953x; 1.0953x over previous
import functools

import jax
import jax.numpy as jnp
from jax.experimental import pallas as pl
from jax.experimental.pallas import tpu as pltpu

_NUM_Q = 4
_COMMIT_W = 0.25


def _rvq_kernel(x_ref, cb_ref, q_ref, idx_ref, loss_ref, *, inv_count):
    i = pl.program_id(0)
    nsteps = pl.num_programs(0)
    r = x_ref[...]
    M, _ = r.shape
    K = cb_ref.shape[1]
    lane_iota = jax.lax.broadcasted_iota(jnp.int32, (M, K), 1)
    loss_part = jnp.float32(0.0)
    idx_cols = []
    for q in range(_NUM_Q):
        cb = cb_ref[q]
        c2 = jnp.sum(cb * cb, axis=1)
        r2 = jnp.sum(r * r, axis=1, keepdims=True)
        scores = jax.lax.dot_general(
            r, cb, (((1,), (1,)), ((), ())),
            precision=None,
            preferred_element_type=jnp.float32)
        d2 = r2 - 2.0 * scores + c2[None, :]
        minv = jnp.min(d2, axis=1, keepdims=True)
        idx = jnp.min(jnp.where(d2 == minv, lane_iota, K), axis=1)
        idx_cols.append(idx[:, None])
        onehot = jnp.where(lane_iota == idx[:, None], 1.0, 0.0)
        quant = jax.lax.dot_general(
            onehot, cb, (((1,), (0,)), ((), ())),
            precision=jax.lax.Precision.HIGHEST,
            preferred_element_type=jnp.float32)
        r = r - quant
        loss_part = loss_part + jnp.sum(r * r)
    q_ref[...] = x_ref[...] - r
    idx_ref[...] = jnp.concatenate(idx_cols, axis=1)

    @pl.when(i == 0)
    def _init():
        loss_ref[0, 0] = jnp.float32(0.0)

    loss_ref[0, 0] += loss_part

    @pl.when(i == nsteps - 1)
    def _finish():
        loss_ref[0, 0] = loss_ref[0, 0] * jnp.float32(_COMMIT_W * inv_count)


def kernel(x, codebooks):
    B, S, D = x.shape
    T = B * S
    K = codebooks.shape[1]
    M = 1152
    xf = x.reshape(T, D)
    qf, idxf, loss = pl.pallas_call(
        functools.partial(_rvq_kernel, inv_count=1.0 / (T * D)),
        grid=(T // M,),
        in_specs=[
            pl.BlockSpec((M, D), lambda i: (i, 0)),
            pl.BlockSpec((_NUM_Q, K, D), lambda i: (0, 0, 0)),
        ],
        out_specs=[
            pl.BlockSpec((M, D), lambda i: (i, 0)),
            pl.BlockSpec((M, _NUM_Q), lambda i: (i, 0)),
            pl.BlockSpec((1, 1), lambda i: (0, 0), memory_space=pltpu.SMEM),
        ],
        out_shape=[
            jax.ShapeDtypeStruct((T, D), jnp.float32),
            jax.ShapeDtypeStruct((T, _NUM_Q), jnp.int32),
            jax.ShapeDtypeStruct((1, 1), jnp.float32),
        ],
    )(xf, codebooks)
    return qf.reshape(B, S, D), idxf.reshape(B, S, _NUM_Q), loss[0, 0]


# speedup vs baseline: 1.7839x; 1.6287x over previous
import functools

import jax
import jax.numpy as jnp
from jax.experimental import pallas as pl
from jax.experimental.pallas import tpu as pltpu

_NUM_Q = 4
_COMMIT_W = 0.25


def _rvq_kernel(x_ref, cb_ref, q_ref, idx_ref, loss_ref,
                hi_ref, mid_ref, lo_ref, *, inv_count):
    i = pl.program_id(0)
    nsteps = pl.num_programs(0)

    @pl.when(i == 0)
    def _split():
        cb = cb_ref[...]
        hi = cb.astype(jnp.bfloat16)
        rem1 = cb - hi.astype(jnp.float32)
        mid = rem1.astype(jnp.bfloat16)
        rem2 = rem1 - mid.astype(jnp.float32)
        hi_ref[...] = hi
        mid_ref[...] = mid
        lo_ref[...] = rem2.astype(jnp.bfloat16)
        loss_ref[0, 0] = jnp.float32(0.0)

    r = x_ref[...]
    M, _ = r.shape
    K = cb_ref.shape[1]
    lane_iota = jax.lax.broadcasted_iota(jnp.int32, (M, K), 1)
    loss_part = jnp.float32(0.0)
    idx_cols = []
    for q in range(_NUM_Q):
        cb = cb_ref[q]
        c2 = jnp.sum(cb * cb, axis=1)
        r2 = jnp.sum(r * r, axis=1, keepdims=True)
        scores = jax.lax.dot_general(
            r, cb, (((1,), (1,)), ((), ())),
            preferred_element_type=jnp.float32)
        d2 = r2 - 2.0 * scores + c2[None, :]
        minv = jnp.min(d2, axis=1, keepdims=True)
        idx = jnp.min(jnp.where(d2 == minv, lane_iota, K), axis=1)
        idx_cols.append(idx[:, None])
        onehot = jnp.where(lane_iota == idx[:, None],
                           jnp.float32(1), jnp.float32(0)).astype(jnp.bfloat16)
        quant = jnp.float32(0.0)
        for part_ref in (hi_ref, mid_ref, lo_ref):
            quant = quant + jax.lax.dot_general(
                onehot, part_ref[q], (((1,), (0,)), ((), ())),
                preferred_element_type=jnp.float32)
        r = r - quant
        loss_part = loss_part + jnp.sum(r * r)
    q_ref[...] = x_ref[...] - r
    idx_ref[...] = jnp.concatenate(idx_cols, axis=1)

    loss_ref[0, 0] += loss_part

    @pl.when(i == nsteps - 1)
    def _finish():
        loss_ref[0, 0] = loss_ref[0, 0] * jnp.float32(_COMMIT_W * inv_count)


def kernel(x, codebooks):
    B, S, D = x.shape
    T = B * S
    K = codebooks.shape[1]
    M = 1152
    xf = x.reshape(T, D)
    qf, idxf, loss = pl.pallas_call(
        functools.partial(_rvq_kernel, inv_count=1.0 / (T * D)),
        grid=(T // M,),
        in_specs=[
            pl.BlockSpec((M, D), lambda i: (i, 0)),
            pl.BlockSpec((_NUM_Q, K, D), lambda i: (0, 0, 0)),
        ],
        out_specs=[
            pl.BlockSpec((M, D), lambda i: (i, 0)),
            pl.BlockSpec((M, _NUM_Q), lambda i: (i, 0)),
            pl.BlockSpec((1, 1), lambda i: (0, 0), memory_space=pltpu.SMEM),
        ],
        out_shape=[
            jax.ShapeDtypeStruct((T, D), jnp.float32),
            jax.ShapeDtypeStruct((T, _NUM_Q), jnp.int32),
            jax.ShapeDtypeStruct((1, 1), jnp.float32),
        ],
        scratch_shapes=[
            pltpu.VMEM((_NUM_Q, K, D), jnp.bfloat16),
            pltpu.VMEM((_NUM_Q, K, D), jnp.bfloat16),
            pltpu.VMEM((_NUM_Q, K, D), jnp.bfloat16),
        ],
    )(xf, codebooks)
    return qf.reshape(B, S, D), idxf.reshape(B, S, _NUM_Q), loss[0, 0]
